# baseline (speedup 1.0000x reference)
import functools

import jax
import jax.numpy as jnp
from jax import lax
from jax.experimental import pallas as pl
from jax.experimental.pallas import tpu as pltpu
from jax.experimental.pallas import tpu_sc as plsc

VOCAB = 5000
VOCAB_PAD = 5008
EMBED = 16
B = 16384
L = 200

NC = 2
NS = 16
NW = NC * NS
RPW = B // NW
G = 16
CH = 128
NCH = RPW // CH
GPC = CH // G


def _tw_body(table_ref, w_ref, b_ref, out_ref):
    out_ref[...] = (
        jnp.sum(table_ref[...] * w_ref[...], axis=1) + b_ref[0, 0]
    )


def _compute_tw(emb_table, W, b):
    table_pad = jnp.zeros((VOCAB_PAD, EMBED), jnp.float32).at[:VOCAB].set(emb_table)
    return pl.pallas_call(
        _tw_body,
        out_shape=jax.ShapeDtypeStruct((VOCAB_PAD,), jnp.float32),
    )(table_pad, W.reshape(1, EMBED), b.reshape(1, 1))


def _sc_body(tw_hbm, idx_hbm, out_hbm, tw_v, idx_v, out_v, part_v, sem0, sem1):
    wid = lax.axis_index("c") * NS + lax.axis_index("s")
    base = wid * RPW
    sems = (sem0, sem1)

    pltpu.sync_copy(tw_hbm, tw_v)
    cps = [
        pltpu.async_copy(idx_hbm.at[pl.ds(base, CH), :], idx_v.at[0], sems[0]),
        None,
    ]

    lane = lax.iota(jnp.int32, 16)
    lane16 = lane * G
    tail_keep = lane >= (G - (L - (L // G) * G))
    cols = [c * G for c in range(L // G)] + [L - G]

    for ch in range(NCH):
        cur = ch & 1
        if ch + 1 < NCH:
            nxt = 1 - cur
            cps[nxt] = pltpu.async_copy(
                idx_hbm.at[pl.ds(base + (ch + 1) * CH, CH), :],
                idx_v.at[nxt],
                sems[nxt],
            )
        cps[cur].wait()
        idx_ch = idx_v.at[cur]

        @plsc.parallel_loop(0, GPC, 1, unroll=1)
        def group(g):
            pbase = g * (G * G)
            for r in range(G):
                row = g * G + r
                acc0 = jnp.zeros((16,), jnp.float32)
                acc1 = jnp.zeros((16,), jnp.float32)
                for i, c in enumerate(cols):
                    tok = idx_ch[row, pl.ds(c, G)]
                    val = plsc.load_gather(tw_v, [tok])
                    if i == len(cols) - 1:
                        val = jnp.where(tail_keep, val, 0.0)
                    if i % 2 == 0:
                        acc0 = acc0 + val
                    else:
                        acc1 = acc1 + val
                plsc.store_scatter(part_v, [pbase + r * G + lane], acc0 + acc1)
            s = jnp.zeros((16,), jnp.float32)
            for c in range(G):
                s = s + plsc.load_gather(part_v, [pbase + lane16 + c])
            m = s * (1.0 / L)
            y = 1.0 / (1.0 + jnp.exp(-m))
            plsc.store_scatter(out_v, [(ch * GPC + g) * G + lane], y)

    pltpu.sync_copy(out_v, out_hbm.at[pl.ds(base, RPW)])


@functools.partial(
    pl.kernel,
    mesh=plsc.VectorSubcoreMesh(core_axis_name="c", subcore_axis_name="s"),
    out_type=jax.ShapeDtypeStruct((B,), jnp.float32),
    scratch_types=[
        pltpu.VMEM((VOCAB_PAD,), jnp.float32),
        pltpu.VMEM((2, CH, L), jnp.int32),
        pltpu.VMEM((RPW,), jnp.float32),
        pltpu.VMEM((GPC * G * G,), jnp.float32),
        pltpu.SemaphoreType.DMA,
        pltpu.SemaphoreType.DMA,
    ],
    compiler_params=pltpu.CompilerParams(needs_layout_passes=False),
)
def _sc_kernel(tw_hbm, idx_hbm, out_hbm, tw_v, idx_v, out_v, part_v, sem0, sem1):
    _sc_body(tw_hbm, idx_hbm, out_hbm, tw_v, idx_v, out_v, part_v, sem0, sem1)


def kernel(inputs, emb_table, W, b):
    tw = _compute_tw(emb_table, W, b)
    out = _sc_kernel(tw, inputs.astype(jnp.int32))
    return out.reshape(B, 1)


# speedup vs baseline: 1.1612x; 1.1612x over previous
import functools

import jax
import jax.numpy as jnp
from jax import lax
from jax.experimental import pallas as pl
from jax.experimental.pallas import tpu as pltpu
from jax.experimental.pallas import tpu_sc as plsc

VOCAB = 5000
VOCAB_PAD = 6144
EMBED = 16
B = 16384
L = 200
VPT = VOCAB_PAD // 16

NC = 2
NS = 16
NW = NC * NS
RPW = B // NW
G = 16
CH = 128
NCH = RPW // CH
GPC = CH // G


def _sc_body(tt_hbm, wb_hbm, idx_hbm, out_hbm, tw_v, tsl_v, wb_v, tws_v,
             shared_v, idx_v, out_v, part_v, sem0, sem1):
    sid = lax.axis_index("s")
    wid = lax.axis_index("c") * NS + sid
    base = wid * RPW
    sems = (sem0, sem1)

    cps = [
        pltpu.async_copy(idx_hbm.at[pl.ds(base, CH), :], idx_v.at[0], sems[0]),
        None,
    ]
    pltpu.sync_copy(tt_hbm.at[:, pl.ds(sid * VPT, VPT)], tsl_v)
    pltpu.sync_copy(wb_hbm, wb_v)
    wvec = wb_v[pl.ds(0, G)]
    bvec = wb_v[pl.ds(G, G)]
    for j in range(VPT // G):
        a0 = jnp.zeros((16,), jnp.float32)
        a1 = bvec
        for e in range(EMBED):
            v = tsl_v[e, pl.ds(j * G, G)] * wvec[e]
            if e % 2 == 0:
                a0 = a0 + v
            else:
                a1 = a1 + v
        tws_v[pl.ds(j * G, G)] = a0 + a1
    pltpu.sync_copy(tws_v, shared_v.at[pl.ds(sid * VPT, VPT)])
    plsc.subcore_barrier()
    pltpu.sync_copy(shared_v, tw_v)

    lane = lax.iota(jnp.int32, 16)
    lane16 = lane * G
    tail_keep = lane >= (G - (L - (L // G) * G))
    cols = [c * G for c in range(L // G)] + [L - G]

    for ch in range(NCH):
        cur = ch & 1
        if ch + 1 < NCH:
            nxt = 1 - cur
            cps[nxt] = pltpu.async_copy(
                idx_hbm.at[pl.ds(base + (ch + 1) * CH, CH), :],
                idx_v.at[nxt],
                sems[nxt],
            )
        cps[cur].wait()
        idx_ch = idx_v.at[cur]

        def group(g, carry):
            for r in range(G):
                row = g * G + r
                acc0 = jnp.zeros((16,), jnp.float32)
                acc1 = jnp.zeros((16,), jnp.float32)
                for i, c in enumerate(cols):
                    tok = idx_ch[row, pl.ds(c, G)]
                    val = plsc.load_gather(tw_v, [tok])
                    if i == len(cols) - 1:
                        val = jnp.where(tail_keep, val, 0.0)
                    if i % 2 == 0:
                        acc0 = acc0 + val
                    else:
                        acc1 = acc1 + val
                part_v[pl.ds(r * G, G)] = acc0 + acc1
            s = jnp.zeros((16,), jnp.float32)
            for c in range(G):
                s = s + plsc.load_gather(part_v, [lane16 + c])
            m = s * (1.0 / L)
            y = 1.0 / (1.0 + jnp.exp(-m))
            plsc.store_scatter(out_v, [(ch * GPC + g) * G + lane], y)
            return carry

        lax.fori_loop(0, GPC, group, 0)

    pltpu.sync_copy(out_v, out_hbm.at[pl.ds(base, RPW)])


@functools.partial(
    pl.kernel,
    mesh=plsc.VectorSubcoreMesh(core_axis_name="c", subcore_axis_name="s"),
    out_type=jax.ShapeDtypeStruct((B,), jnp.float32),
    scratch_types=[
        pltpu.VMEM((VOCAB_PAD,), jnp.float32),
        pltpu.VMEM((EMBED, VPT), jnp.float32),
        pltpu.VMEM((2 * G,), jnp.float32),
        pltpu.VMEM((VPT,), jnp.float32),
        pltpu.VMEM_SHARED((VOCAB_PAD,), jnp.float32),
        pltpu.VMEM((2, CH, L), jnp.int32),
        pltpu.VMEM((RPW,), jnp.float32),
        pltpu.VMEM((G * G,), jnp.float32),
        pltpu.SemaphoreType.DMA,
        pltpu.SemaphoreType.DMA,
    ],
    compiler_params=pltpu.CompilerParams(needs_layout_passes=False),
)
def _sc_kernel(tt_hbm, wb_hbm, idx_hbm, out_hbm, tw_v, tsl_v, wb_v, tws_v,
               shared_v, idx_v, out_v, part_v, sem0, sem1):
    _sc_body(tt_hbm, wb_hbm, idx_hbm, out_hbm, tw_v, tsl_v, wb_v, tws_v,
             shared_v, idx_v, out_v, part_v, sem0, sem1)


def kernel(inputs, emb_table, W, b):
    tt = jnp.zeros((EMBED, VOCAB_PAD), jnp.float32).at[:, :VOCAB].set(emb_table.T)
    wb = jnp.concatenate([W.reshape(EMBED), jnp.broadcast_to(b, (G,))])
    out = _sc_kernel(tt, wb, inputs.astype(jnp.int32))
    return out.reshape(B, 1)
